# baseline (speedup 1.0000x reference)
import functools

import jax
import jax.numpy as jnp
from jax.experimental import pallas as pl
from jax.experimental.pallas import tpu as pltpu

_N_NODES = 100000
_D = 128
_N_DST1 = 36864
_N_DST2 = 4096


def _dense_body(relu, hd_ref, sm_ref, rd_ref, ws_ref, wn_ref, b_ref, o_ref):
    hn = sm_ref[...] * rd_ref[...]
    acc = jnp.dot(hd_ref[...], ws_ref[...], preferred_element_type=jnp.float32)
    acc += jnp.dot(hn, wn_ref[...], preferred_element_type=jnp.float32)
    acc += b_ref[...]
    if relu:
        acc = jnp.maximum(acc, 0.0)
    o_ref[...] = acc


def _dense_layer(h_dst, summed, rdeg, W_self, W_neigh, b, relu, block_m):
    m = h_dst.shape[0]
    grid = (m // block_m,)
    return pl.pallas_call(
        functools.partial(_dense_body, relu),
        grid=grid,
        in_specs=[
            pl.BlockSpec((block_m, _D), lambda i: (i, 0)),
            pl.BlockSpec((block_m, _D), lambda i: (i, 0)),
            pl.BlockSpec((block_m, 1), lambda i: (i, 0)),
            pl.BlockSpec((_D, _D), lambda i: (0, 0)),
            pl.BlockSpec((_D, _D), lambda i: (0, 0)),
            pl.BlockSpec((1, _D), lambda i: (0, 0)),
        ],
        out_specs=pl.BlockSpec((block_m, _D), lambda i: (i, 0)),
        out_shape=jax.ShapeDtypeStruct((m, _D), jnp.float32),
    )(h_dst, summed, rdeg, W_self, W_neigh, b.reshape(1, _D))


def kernel(input_nodes, edge_src1, edge_dst1, edge_src2, edge_dst2, embed,
           W_self1, W_neigh1, b1, W_self2, W_neigh2, b2):
    ci1 = jnp.take(input_nodes, edge_src1, axis=0)
    msg = jnp.take(embed, ci1, axis=0)
    summed1 = jax.ops.segment_sum(msg, edge_dst1, num_segments=_N_DST1)
    deg1 = jax.ops.segment_sum(jnp.ones(edge_dst1.shape, jnp.float32),
                               edge_dst1, num_segments=_N_DST1)
    rdeg1 = (1.0 / jnp.clip(deg1, 1.0, None)).reshape(_N_DST1, 1)
    h_dst1 = jnp.take(embed, input_nodes[:_N_DST1], axis=0)
    out1 = _dense_layer(h_dst1, summed1, rdeg1, W_self1, W_neigh1, b1,
                        relu=True, block_m=512)

    msg2 = jnp.take(out1, edge_src2, axis=0)
    summed2 = jax.ops.segment_sum(msg2, edge_dst2, num_segments=_N_DST2)
    deg2 = jax.ops.segment_sum(jnp.ones(edge_dst2.shape, jnp.float32),
                               edge_dst2, num_segments=_N_DST2)
    rdeg2 = (1.0 / jnp.clip(deg2, 1.0, None)).reshape(_N_DST2, 1)
    out = _dense_layer(out1[:_N_DST2], summed2, rdeg2, W_self2, W_neigh2, b2,
                       relu=False, block_m=512)
    return out


# speedup vs baseline: 3.6726x; 3.6726x over previous
import functools

import jax
import jax.numpy as jnp
from jax import lax
from jax.experimental import pallas as pl
from jax.experimental.pallas import tpu as pltpu
from jax.experimental.pallas import tpu_sc as plsc

_D = 128
_N_NODES = 100000
_N_DST1 = 36864
_N_E1 = 294912
_N_DST2 = 4096
_N_E2 = 32768

_NC = 2
_NS = 16
_L = 16
_CH = 64
_CHS = 6
_B = 2048

_EW1 = _N_E1 // _NS
_NB1 = _EW1 // _B
_HALF1 = _N_DST1 // _NC
_NPASS1 = 2
_RNG1 = _HALF1 // _NPASS1
_ACC1 = 9344
_DR1 = _RNG1 // _NS
_HD_W = _N_DST1 // (_NC * _NS)

_EW2 = _N_E2 // _NS
_HALF2 = _N_DST2 // _NC
_ACC2 = 2176
_DR2 = _HALF2 // _NS

_mesh = plsc.VectorSubcoreMesh(core_axis_name="c", subcore_axis_name="s")
_sc_params = pltpu.CompilerParams(needs_layout_passes=False)


def _mask_block(dstbuf, dbuf2, lo, hi, trash):
    for r in range(_B // _CH):
        for j in range(_CH // _L):
            dv = dstbuf[pl.ds(r * _CH + j * _L, _L)]
            m = jnp.logical_and(dv >= lo, dv < hi)
            dbuf2[r, pl.ds(j * _L, _L)] = jnp.where(m, dv - lo, trash)


def _gather_scatter_block(table_hbm, cibuf2d, cirow0, dbuf2, rows_v, onesv,
                          acc, dega):
    for ch in range(_B // _CH):
        pltpu.sync_copy(table_hbm.at[cibuf2d.at[cirow0 + ch]], rows_v)
        pltpu.sync_copy(rows_v, acc.at[dbuf2.at[ch]], add=True)
        pltpu.sync_copy(onesv, dega.at[dbuf2.at[ch]], add=True)


def _zero_acc(zbuf, degzv, acc, dega, s):
    zw = dega.shape[0] // _NS
    base = s * zw
    for j in range(zw // _L):
        pltpu.sync_copy(zbuf, acc.at[pl.ds(base + j * _L, _L)])
    rem = zw - (zw // _L) * _L
    if rem:
        pltpu.sync_copy(zbuf.at[pl.ds(0, rem)],
                        acc.at[pl.ds(base + zw - rem, rem)])
    pltpu.sync_copy(degzv.at[pl.ds(0, zw)], dega.at[pl.ds(base, zw)])


def _drain(acc, dega, rows_v, degv, summed_hbm, deg_hbm, lo, s, dr):
    base = s * dr
    for k in range(dr // _CH):
        pltpu.sync_copy(acc.at[pl.ds(base + k * _CH, _CH)], rows_v)
        pltpu.sync_copy(rows_v, summed_hbm.at[pl.ds(lo + base + k * _CH, _CH)])
    rem = dr - (dr // _CH) * _CH
    if rem:
        pltpu.sync_copy(acc.at[pl.ds(base + dr - rem, rem)],
                        rows_v.at[pl.ds(0, rem)])
        pltpu.sync_copy(rows_v.at[pl.ds(0, rem)],
                        summed_hbm.at[pl.ds(lo + base + dr - rem, rem)])
    pltpu.sync_copy(dega.at[pl.ds(base, dr)], degv.at[pl.ds(0, dr)])
    pltpu.sync_copy(degv.at[pl.ds(0, dr)], deg_hbm.at[pl.ds(lo + base, dr)])


def _sc1_body(nodes_hbm, src3d_hbm, dst_hbm, embed_hbm, nodes2d_hbm,
              zeros_hbm, zeros1_hbm,
              ones_hbm, summed_hbm, deg_hbm, hdst_hbm,
              cibuf, dstbuf, dbuf2, rows_v, onesv, zbuf,
              degv, acc, dega):
    c = lax.axis_index("c")
    s = lax.axis_index("s")

    pltpu.sync_copy(zeros_hbm, zbuf)
    pltpu.sync_copy(ones_hbm, onesv)
    pltpu.sync_copy(zeros1_hbm, degv)
    ebase = s * _EW1
    for b in range(_NB1):
        pltpu.sync_copy(src3d_hbm.at[s * _NB1 + b], dbuf2)
        for r in range(_B // _CH):
            pltpu.sync_copy(nodes_hbm.at[dbuf2.at[r]],
                            cibuf.at[b * (_B // _CH) + r])

    for p in range(_NPASS1):
        lo = c * _HALF1 + p * _RNG1
        _zero_acc(zbuf, degv, acc, dega, s)
        plsc.subcore_barrier()

        def pass_body(b, _, lo=lo):
            pltpu.sync_copy(dst_hbm.at[pl.ds(ebase + b * _B, _B)], dstbuf)
            _mask_block(dstbuf, dbuf2, lo, lo + _RNG1, _RNG1)
            _gather_scatter_block(embed_hbm, cibuf, b * (_B // _CH), dbuf2,
                                  rows_v, onesv, acc, dega)
            return 0

        lax.fori_loop(0, _NB1, pass_body, 0)
        plsc.subcore_barrier()
        _drain(acc, dega, rows_v, degv, summed_hbm, deg_hbm, lo, s, _DR1)
        pltpu.sync_copy(zeros1_hbm, degv)
        plsc.subcore_barrier()

    wid = c * _NS + s
    nrow = _HD_W // _CH
    pltpu.sync_copy(nodes2d_hbm.at[wid], dbuf2.at[pl.ds(0, nrow)])
    for k in range(nrow):
        pltpu.sync_copy(embed_hbm.at[dbuf2.at[k]], rows_v)
        pltpu.sync_copy(rows_v,
                        hdst_hbm.at[pl.ds(wid * _HD_W + k * _CH, _CH)])


def _sc1_call(input_nodes, edge_src1, edge_dst1, embed):
    zeros = jnp.zeros((_L, _D), jnp.float32)
    zeros1 = jnp.zeros((_ACC1 // _NS,), jnp.float32)
    ones = jnp.ones((_CH,), jnp.float32)
    src3d = edge_src1.reshape(_NS * _NB1, _B // _CH, _CH)
    nodes2d = input_nodes[:_N_DST1].reshape(_NC * _NS, _HD_W // _CH, _CH)
    f = pl.kernel(
        _sc1_body,
        out_type=(
            jax.ShapeDtypeStruct((_N_DST1, _D), jnp.float32),
            jax.ShapeDtypeStruct((_N_DST1,), jnp.float32),
            jax.ShapeDtypeStruct((_N_DST1, _D), jnp.float32),
        ),
        mesh=_mesh,
        scratch_types=[
            pltpu.VMEM((_EW1 // _CH, _CH), jnp.int32),
            pltpu.VMEM((_B,), jnp.int32),
            pltpu.VMEM((_B // _CH, _CH), jnp.int32),
            pltpu.VMEM((_CH, _D), jnp.float32),
            pltpu.VMEM((_CH,), jnp.float32),
            pltpu.VMEM((_L, _D), jnp.float32),
            pltpu.VMEM((_ACC1 // _NS,), jnp.float32),
            pltpu.VMEM_SHARED((_ACC1, _D), jnp.float32),
            pltpu.VMEM_SHARED((_ACC1,), jnp.float32),
        ],
        compiler_params=_sc_params,
    )
    return f(input_nodes, src3d, edge_dst1, embed, nodes2d, zeros, zeros1,
             ones)


def _sc2_body(h_hbm, src3d_hbm, dst_hbm, zeros_hbm, zeros1_hbm, ones_hbm,
              summed_hbm, deg_hbm,
              srcbuf, dstbuf, dbuf2, rows_v, onesv,
              zbuf, degv, acc, dega):
    c = lax.axis_index("c")
    s = lax.axis_index("s")

    pltpu.sync_copy(zeros_hbm, zbuf)
    pltpu.sync_copy(ones_hbm, onesv)
    pltpu.sync_copy(zeros1_hbm, degv)
    pltpu.sync_copy(dst_hbm.at[pl.ds(s * _EW2, _EW2)], dstbuf)
    pltpu.sync_copy(src3d_hbm.at[s], srcbuf)

    lo = c * _HALF2
    _zero_acc(zbuf, degv, acc, dega, s)
    plsc.subcore_barrier()
    _mask_block(dstbuf, dbuf2, lo, lo + _HALF2, _HALF2)
    _gather_scatter_block(h_hbm, srcbuf, 0, dbuf2, rows_v, onesv, acc, dega)
    plsc.subcore_barrier()
    _drain(acc, dega, rows_v, degv, summed_hbm, deg_hbm, lo, s, _DR2)


def _sc2_call(h1, edge_src2, edge_dst2):
    zeros = jnp.zeros((_L, _D), jnp.float32)
    zeros1 = jnp.zeros((_ACC2 // _NS,), jnp.float32)
    ones = jnp.ones((_CH,), jnp.float32)
    f = pl.kernel(
        _sc2_body,
        out_type=(
            jax.ShapeDtypeStruct((_N_DST2, _D), jnp.float32),
            jax.ShapeDtypeStruct((_N_DST2,), jnp.float32),
        ),
        mesh=_mesh,
        scratch_types=[
            pltpu.VMEM((_EW2 // _CH, _CH), jnp.int32),
            pltpu.VMEM((_EW2,), jnp.int32),
            pltpu.VMEM((_B // _CH, _CH), jnp.int32),
            pltpu.VMEM((_CH, _D), jnp.float32),
            pltpu.VMEM((_CH,), jnp.float32),
            pltpu.VMEM((_L, _D), jnp.float32),
            pltpu.VMEM((_ACC2 // _NS,), jnp.float32),
            pltpu.VMEM_SHARED((_ACC2, _D), jnp.float32),
            pltpu.VMEM_SHARED((_ACC2,), jnp.float32),
        ],
        compiler_params=_sc_params,
    )
    src3d = edge_src2.reshape(_NS, _EW2 // _CH, _CH)
    return f(h1, src3d, edge_dst2, zeros, zeros1, ones)


def _dense_body(relu, hd_ref, sm_ref, dg_ref, ws_ref, wn_ref, b_ref, o_ref):
    rd = 1.0 / jnp.maximum(dg_ref[...], 1.0)
    hn = sm_ref[...] * rd
    acc = jnp.dot(hd_ref[...], ws_ref[...], preferred_element_type=jnp.float32)
    acc += jnp.dot(hn, wn_ref[...], preferred_element_type=jnp.float32)
    acc += b_ref[...]
    if relu:
        acc = jnp.maximum(acc, 0.0)
    o_ref[...] = acc


def _dense_layer(h_dst, summed, deg, W_self, W_neigh, b, relu, block_m):
    m = h_dst.shape[0]
    grid = (m // block_m,)
    return pl.pallas_call(
        functools.partial(_dense_body, relu),
        grid=grid,
        in_specs=[
            pl.BlockSpec((block_m, _D), lambda i: (i, 0)),
            pl.BlockSpec((block_m, _D), lambda i: (i, 0)),
            pl.BlockSpec((block_m, 1), lambda i: (i, 0)),
            pl.BlockSpec((_D, _D), lambda i: (0, 0)),
            pl.BlockSpec((_D, _D), lambda i: (0, 0)),
            pl.BlockSpec((1, _D), lambda i: (0, 0)),
        ],
        out_specs=pl.BlockSpec((block_m, _D), lambda i: (i, 0)),
        out_shape=jax.ShapeDtypeStruct((m, _D), jnp.float32),
    )(h_dst, summed, deg.reshape(m, 1), W_self, W_neigh, b.reshape(1, _D))


def kernel(input_nodes, edge_src1, edge_dst1, edge_src2, edge_dst2, embed,
           W_self1, W_neigh1, b1, W_self2, W_neigh2, b2):
    summed1, deg1, h_dst1 = _sc1_call(input_nodes, edge_src1, edge_dst1, embed)
    out1 = _dense_layer(h_dst1, summed1, deg1, W_self1, W_neigh1, b1,
                        relu=True, block_m=512)
    summed2, deg2 = _sc2_call(out1, edge_src2, edge_dst2)
    out = _dense_layer(out1[:_N_DST2], summed2, deg2, W_self2, W_neigh2, b2,
                       relu=False, block_m=512)
    return out


# speedup vs baseline: 4.0100x; 1.0919x over previous
import functools

import jax
import jax.numpy as jnp
from jax import lax
from jax.experimental import pallas as pl
from jax.experimental.pallas import tpu as pltpu
from jax.experimental.pallas import tpu_sc as plsc

_D = 128
_N_NODES = 100000
_N_DST1 = 36864
_N_E1 = 294912
_N_DST2 = 4096
_N_E2 = 32768

_NC = 2
_NS = 16
_L = 16
_CH = 64
_CHS = 6
_B = 2048

_EW1 = _N_E1 // _NS
_NB1 = _EW1 // _B
_HALF1 = _N_DST1 // _NC
_NPASS1 = 2
_RNG1 = _HALF1 // _NPASS1
_ACC1 = 9344
_DR1 = _RNG1 // _NS
_HD_W = _N_DST1 // (_NC * _NS)

_EW2 = _N_E2 // _NS
_HALF2 = _N_DST2 // _NC
_ACC2 = 2176
_DR2 = _HALF2 // _NS

_mesh = plsc.VectorSubcoreMesh(core_axis_name="c", subcore_axis_name="s")
_sc_params = pltpu.CompilerParams(needs_layout_passes=False)


def _mask_block(dstbuf, dbuf2, lo, hi, trash):
    for r in range(_B // _CH):
        for j in range(_CH // _L):
            dv = dstbuf[pl.ds(r * _CH + j * _L, _L)]
            m = jnp.logical_and(dv >= lo, dv < hi)
            dbuf2[r, pl.ds(j * _L, _L)] = jnp.where(m, dv - lo, trash)


def _gather_scatter_block(table_hbm, cibuf2d, cirow0, dbuf2, rows_v, rows_w,
                          sem_a, sem_b, onesv, acc, dega):
    rows = (rows_v, rows_w)
    sems = (sem_a, sem_b)
    nchb = _B // _CH
    cp = pltpu.async_copy(table_hbm.at[cibuf2d.at[cirow0]], rows[0], sems[0])
    for ch in range(nchb):
        nxt = ch + 1
        ncp = None
        if nxt < nchb:
            ncp = pltpu.async_copy(table_hbm.at[cibuf2d.at[cirow0 + nxt]],
                                   rows[nxt % 2], sems[nxt % 2])
        cp.wait()
        pltpu.sync_copy(rows[ch % 2], acc.at[dbuf2.at[ch]], add=True)
        pltpu.sync_copy(onesv, dega.at[dbuf2.at[ch]], add=True)
        cp = ncp


def _zero_acc(zbuf, degzv, acc, dega, s):
    zw = dega.shape[0] // _NS
    base = s * zw
    for j in range(zw // _L):
        pltpu.sync_copy(zbuf, acc.at[pl.ds(base + j * _L, _L)])
    rem = zw - (zw // _L) * _L
    if rem:
        pltpu.sync_copy(zbuf.at[pl.ds(0, rem)],
                        acc.at[pl.ds(base + zw - rem, rem)])
    pltpu.sync_copy(degzv.at[pl.ds(0, zw)], dega.at[pl.ds(base, zw)])


def _drain(acc, dega, rows_v, degv, summed_hbm, deg_hbm, lo, s, dr):
    base = s * dr
    for k in range(dr // _CH):
        pltpu.sync_copy(acc.at[pl.ds(base + k * _CH, _CH)], rows_v)
        pltpu.sync_copy(rows_v, summed_hbm.at[pl.ds(lo + base + k * _CH, _CH)])
    rem = dr - (dr // _CH) * _CH
    if rem:
        pltpu.sync_copy(acc.at[pl.ds(base + dr - rem, rem)],
                        rows_v.at[pl.ds(0, rem)])
        pltpu.sync_copy(rows_v.at[pl.ds(0, rem)],
                        summed_hbm.at[pl.ds(lo + base + dr - rem, rem)])
    pltpu.sync_copy(dega.at[pl.ds(base, dr)], degv.at[pl.ds(0, dr)])
    pltpu.sync_copy(degv.at[pl.ds(0, dr)], deg_hbm.at[pl.ds(lo + base, dr)])


def _sc1_body(nodes_hbm, src3d_hbm, dst_hbm, embed_hbm, nodes2d_hbm,
              zeros_hbm, zeros1_hbm,
              ones_hbm, summed_hbm, deg_hbm, hdst_hbm, ci3d_hbm,
              ciblk, dstbuf, dbuf2, rows_v, rows_w, onesv, zbuf,
              degv, acc, dega, sem_a, sem_b):
    c = lax.axis_index("c")
    s = lax.axis_index("s")

    pltpu.sync_copy(zeros_hbm, zbuf)
    pltpu.sync_copy(ones_hbm, onesv)
    pltpu.sync_copy(zeros1_hbm, degv)
    ebase = s * _EW1
    nchb = _B // _CH
    for b in range(_NB1):
        pltpu.sync_copy(src3d_hbm.at[s * _NB1 + b], dbuf2)
        cp = pltpu.async_copy(nodes_hbm.at[dbuf2.at[0]], ciblk.at[0], sem_a)
        for r in range(nchb):
            ncp = None
            if r + 1 < nchb:
                ncp = pltpu.async_copy(nodes_hbm.at[dbuf2.at[r + 1]],
                                       ciblk.at[r + 1],
                                       sem_b if (r + 1) % 2 else sem_a)
            cp.wait()
            cp = ncp
        pltpu.sync_copy(ciblk, ci3d_hbm.at[s * _NB1 + b])

    for p in range(_NPASS1):
        lo = c * _HALF1 + p * _RNG1
        _zero_acc(zbuf, degv, acc, dega, s)
        plsc.subcore_barrier()

        def pass_body(b, _, lo=lo):
            pltpu.sync_copy(dst_hbm.at[pl.ds(ebase + b * _B, _B)], dstbuf)
            pltpu.sync_copy(ci3d_hbm.at[s * _NB1 + b], ciblk)
            _mask_block(dstbuf, dbuf2, lo, lo + _RNG1, _RNG1)
            _gather_scatter_block(embed_hbm, ciblk, 0, dbuf2,
                                  rows_v, rows_w, sem_a, sem_b, onesv,
                                  acc, dega)
            return 0

        lax.fori_loop(0, _NB1, pass_body, 0)
        plsc.subcore_barrier()
        _drain(acc, dega, rows_v, degv, summed_hbm, deg_hbm, lo, s, _DR1)
        pltpu.sync_copy(zeros1_hbm, degv)
        plsc.subcore_barrier()

    wid = c * _NS + s
    nrow = _HD_W // _CH
    pltpu.sync_copy(nodes2d_hbm.at[wid], dbuf2.at[pl.ds(0, nrow)])
    for k in range(nrow):
        pltpu.sync_copy(embed_hbm.at[dbuf2.at[k]], rows_v)
        pltpu.sync_copy(rows_v,
                        hdst_hbm.at[pl.ds(wid * _HD_W + k * _CH, _CH)])


def _sc1_call(input_nodes, edge_src1, edge_dst1, embed):
    zeros = jnp.zeros((_L, _D), jnp.float32)
    zeros1 = jnp.zeros((_ACC1 // _NS,), jnp.float32)
    ones = jnp.ones((_CH,), jnp.float32)
    src3d = edge_src1.reshape(_NS * _NB1, _B // _CH, _CH)
    nodes2d = input_nodes[:_N_DST1].reshape(_NC * _NS, _HD_W // _CH, _CH)
    f = pl.kernel(
        _sc1_body,
        out_type=(
            jax.ShapeDtypeStruct((_N_DST1, _D), jnp.float32),
            jax.ShapeDtypeStruct((_N_DST1,), jnp.float32),
            jax.ShapeDtypeStruct((_N_DST1, _D), jnp.float32),
            jax.ShapeDtypeStruct((_NS * _NB1, _B // _CH, _CH), jnp.int32),
        ),
        mesh=_mesh,
        scratch_types=[
            pltpu.VMEM((_B // _CH, _CH), jnp.int32),
            pltpu.VMEM((_B,), jnp.int32),
            pltpu.VMEM((_B // _CH, _CH), jnp.int32),
            pltpu.VMEM((_CH, _D), jnp.float32),
            pltpu.VMEM((_CH, _D), jnp.float32),
            pltpu.VMEM((_CH,), jnp.float32),
            pltpu.VMEM((_L, _D), jnp.float32),
            pltpu.VMEM((_ACC1 // _NS,), jnp.float32),
            pltpu.VMEM_SHARED((_ACC1, _D), jnp.float32),
            pltpu.VMEM_SHARED((_ACC1,), jnp.float32),
            pltpu.SemaphoreType.DMA,
            pltpu.SemaphoreType.DMA,
        ],
        compiler_params=_sc_params,
    )
    return f(input_nodes, src3d, edge_dst1, embed, nodes2d, zeros, zeros1,
             ones)


def _sc2_body(h_hbm, src3d_hbm, dst_hbm, zeros_hbm, zeros1_hbm, ones_hbm,
              summed_hbm, deg_hbm,
              srcbuf, dstbuf, dbuf2, rows_v, rows_w, onesv,
              zbuf, degv, acc, dega, sem_a, sem_b):
    c = lax.axis_index("c")
    s = lax.axis_index("s")

    pltpu.sync_copy(zeros_hbm, zbuf)
    pltpu.sync_copy(ones_hbm, onesv)
    pltpu.sync_copy(zeros1_hbm, degv)
    pltpu.sync_copy(dst_hbm.at[pl.ds(s * _EW2, _EW2)], dstbuf)
    pltpu.sync_copy(src3d_hbm.at[s], srcbuf)

    lo = c * _HALF2
    _zero_acc(zbuf, degv, acc, dega, s)
    plsc.subcore_barrier()
    _mask_block(dstbuf, dbuf2, lo, lo + _HALF2, _HALF2)
    _gather_scatter_block(h_hbm, srcbuf, 0, dbuf2, rows_v, rows_w,
                          sem_a, sem_b, onesv, acc, dega)
    plsc.subcore_barrier()
    _drain(acc, dega, rows_v, degv, summed_hbm, deg_hbm, lo, s, _DR2)


def _sc2_call(h1, edge_src2, edge_dst2):
    zeros = jnp.zeros((_L, _D), jnp.float32)
    zeros1 = jnp.zeros((_ACC2 // _NS,), jnp.float32)
    ones = jnp.ones((_CH,), jnp.float32)
    f = pl.kernel(
        _sc2_body,
        out_type=(
            jax.ShapeDtypeStruct((_N_DST2, _D), jnp.float32),
            jax.ShapeDtypeStruct((_N_DST2,), jnp.float32),
        ),
        mesh=_mesh,
        scratch_types=[
            pltpu.VMEM((_EW2 // _CH, _CH), jnp.int32),
            pltpu.VMEM((_EW2,), jnp.int32),
            pltpu.VMEM((_B // _CH, _CH), jnp.int32),
            pltpu.VMEM((_CH, _D), jnp.float32),
            pltpu.VMEM((_CH, _D), jnp.float32),
            pltpu.VMEM((_CH,), jnp.float32),
            pltpu.VMEM((_L, _D), jnp.float32),
            pltpu.VMEM((_ACC2 // _NS,), jnp.float32),
            pltpu.VMEM_SHARED((_ACC2, _D), jnp.float32),
            pltpu.VMEM_SHARED((_ACC2,), jnp.float32),
            pltpu.SemaphoreType.DMA,
            pltpu.SemaphoreType.DMA,
        ],
        compiler_params=_sc_params,
    )
    src3d = edge_src2.reshape(_NS, _EW2 // _CH, _CH)
    return f(h1, src3d, edge_dst2, zeros, zeros1, ones)


def _dense_body(relu, hd_ref, sm_ref, dg_ref, ws_ref, wn_ref, b_ref, o_ref):
    rd = 1.0 / jnp.maximum(dg_ref[...], 1.0)
    hn = sm_ref[...] * rd
    acc = jnp.dot(hd_ref[...], ws_ref[...], preferred_element_type=jnp.float32)
    acc += jnp.dot(hn, wn_ref[...], preferred_element_type=jnp.float32)
    acc += b_ref[...]
    if relu:
        acc = jnp.maximum(acc, 0.0)
    o_ref[...] = acc


def _dense_layer(h_dst, summed, deg, W_self, W_neigh, b, relu, block_m):
    m = h_dst.shape[0]
    grid = (m // block_m,)
    return pl.pallas_call(
        functools.partial(_dense_body, relu),
        grid=grid,
        in_specs=[
            pl.BlockSpec((block_m, _D), lambda i: (i, 0)),
            pl.BlockSpec((block_m, _D), lambda i: (i, 0)),
            pl.BlockSpec((block_m, 1), lambda i: (i, 0)),
            pl.BlockSpec((_D, _D), lambda i: (0, 0)),
            pl.BlockSpec((_D, _D), lambda i: (0, 0)),
            pl.BlockSpec((1, _D), lambda i: (0, 0)),
        ],
        out_specs=pl.BlockSpec((block_m, _D), lambda i: (i, 0)),
        out_shape=jax.ShapeDtypeStruct((m, _D), jnp.float32),
    )(h_dst, summed, deg.reshape(m, 1), W_self, W_neigh, b.reshape(1, _D))


def kernel(input_nodes, edge_src1, edge_dst1, edge_src2, edge_dst2, embed,
           W_self1, W_neigh1, b1, W_self2, W_neigh2, b2):
    summed1, deg1, h_dst1, _ = _sc1_call(input_nodes, edge_src1, edge_dst1,
                                         embed)
    out1 = _dense_layer(h_dst1, summed1, deg1, W_self1, W_neigh1, b1,
                        relu=True, block_m=512)
    summed2, deg2 = _sc2_call(out1, edge_src2, edge_dst2)
    out = _dense_layer(out1[:_N_DST2], summed2, deg2, W_self2, W_neigh2, b2,
                       relu=False, block_m=512)
    return out


# speedup vs baseline: 4.0140x; 1.0010x over previous
import functools

import jax
import jax.numpy as jnp
from jax import lax
from jax.experimental import pallas as pl
from jax.experimental.pallas import tpu as pltpu
from jax.experimental.pallas import tpu_sc as plsc

_D = 128
_N_NODES = 100000
_N_DST1 = 36864
_N_E1 = 294912
_N_DST2 = 4096
_N_E2 = 32768

_NC = 2
_NS = 16
_L = 16
_CH = 64
_CHS = 6
_B = 2048

_EW1 = _N_E1 // _NS
_NB1 = _EW1 // _B
_HALF1 = _N_DST1 // _NC
_NPASS1 = 2
_RNG1 = _HALF1 // _NPASS1
_ACC1 = 9344
_DR1 = _RNG1 // _NS
_HD_W = _N_DST1 // (_NC * _NS)

_EW2 = _N_E2 // _NS
_HALF2 = _N_DST2 // _NC
_ACC2 = 2176
_DR2 = _HALF2 // _NS

_mesh = plsc.VectorSubcoreMesh(core_axis_name="c", subcore_axis_name="s")
_sc_params = pltpu.CompilerParams(needs_layout_passes=False)


def _mask_block(dstbuf, dbuf2, lo, hi, trash):
    for r in range(_B // _CH):
        for j in range(_CH // _L):
            dv = dstbuf[pl.ds(r * _CH + j * _L, _L)]
            m = jnp.logical_and(dv >= lo, dv < hi)
            dbuf2[r, pl.ds(j * _L, _L)] = jnp.where(m, dv - lo, trash)


def _gather_scatter_block(table_hbm, cibuf2d, cirow0, dbuf2, rows_v, rows_w,
                          sem_a, sem_b, sem_c, sem_d, sem_e, onesv,
                          acc, dega):
    rows = (rows_v, rows_w)
    gsem = (sem_a, sem_b)
    ssem = (sem_c, sem_d)
    nchb = _B // _CH
    gl = [
        pltpu.async_copy(table_hbm.at[cibuf2d.at[cirow0]], rows[0], gsem[0]),
        pltpu.async_copy(table_hbm.at[cibuf2d.at[cirow0 + 1]], rows[1],
                         gsem[1]),
    ]
    scs = [None] * nchb
    dcs = []
    for ch in range(nchb):
        gl[ch % 2].wait()
        scs[ch] = pltpu.async_copy(rows[ch % 2], acc.at[dbuf2.at[ch]],
                                   ssem[ch % 2], add=True)
        dcs.append(pltpu.async_copy(onesv, dega.at[dbuf2.at[ch]], sem_e,
                                    add=True))
        if ch + 2 < nchb:
            scs[ch].wait()
            gl[ch % 2] = pltpu.async_copy(
                table_hbm.at[cibuf2d.at[cirow0 + ch + 2]], rows[ch % 2],
                gsem[ch % 2])
    scs[nchb - 2].wait()
    scs[nchb - 1].wait()
    for cp in dcs:
        cp.wait()


def _zero_acc(zbuf, degzv, acc, dega, s):
    zw = dega.shape[0] // _NS
    base = s * zw
    for j in range(zw // _L):
        pltpu.sync_copy(zbuf, acc.at[pl.ds(base + j * _L, _L)])
    rem = zw - (zw // _L) * _L
    if rem:
        pltpu.sync_copy(zbuf.at[pl.ds(0, rem)],
                        acc.at[pl.ds(base + zw - rem, rem)])
    pltpu.sync_copy(degzv.at[pl.ds(0, zw)], dega.at[pl.ds(base, zw)])


def _drain(acc, dega, rows_v, degv, summed_hbm, deg_hbm, lo, s, dr):
    base = s * dr
    for k in range(dr // _CH):
        pltpu.sync_copy(acc.at[pl.ds(base + k * _CH, _CH)], rows_v)
        pltpu.sync_copy(rows_v, summed_hbm.at[pl.ds(lo + base + k * _CH, _CH)])
    rem = dr - (dr // _CH) * _CH
    if rem:
        pltpu.sync_copy(acc.at[pl.ds(base + dr - rem, rem)],
                        rows_v.at[pl.ds(0, rem)])
        pltpu.sync_copy(rows_v.at[pl.ds(0, rem)],
                        summed_hbm.at[pl.ds(lo + base + dr - rem, rem)])
    pltpu.sync_copy(dega.at[pl.ds(base, dr)], degv.at[pl.ds(0, dr)])
    pltpu.sync_copy(degv.at[pl.ds(0, dr)], deg_hbm.at[pl.ds(lo + base, dr)])


def _sc1_body(nodes_hbm, src3d_hbm, dst_hbm, embed_hbm, nodes2d_hbm,
              zeros_hbm, zeros1_hbm,
              ones_hbm, summed_hbm, deg_hbm, hdst_hbm, ci3d_hbm,
              ciblk, dstbuf, dbuf2, rows_v, rows_w, onesv, zbuf,
              degv, acc, dega, sem_a, sem_b, sem_c, sem_d, sem_e):
    c = lax.axis_index("c")
    s = lax.axis_index("s")

    pltpu.sync_copy(zeros_hbm, zbuf)
    pltpu.sync_copy(ones_hbm, onesv)
    pltpu.sync_copy(zeros1_hbm, degv)
    ebase = s * _EW1
    nchb = _B // _CH
    for b in range(_NB1):
        pltpu.sync_copy(src3d_hbm.at[s * _NB1 + b], dbuf2)
        cp = pltpu.async_copy(nodes_hbm.at[dbuf2.at[0]], ciblk.at[0], sem_a)
        for r in range(nchb):
            ncp = None
            if r + 1 < nchb:
                ncp = pltpu.async_copy(nodes_hbm.at[dbuf2.at[r + 1]],
                                       ciblk.at[r + 1],
                                       sem_b if (r + 1) % 2 else sem_a)
            cp.wait()
            cp = ncp
        pltpu.sync_copy(ciblk, ci3d_hbm.at[s * _NB1 + b])

    for p in range(_NPASS1):
        lo = c * _HALF1 + p * _RNG1
        _zero_acc(zbuf, degv, acc, dega, s)
        plsc.subcore_barrier()

        def pass_body(b, _, lo=lo):
            pltpu.sync_copy(dst_hbm.at[pl.ds(ebase + b * _B, _B)], dstbuf)
            pltpu.sync_copy(ci3d_hbm.at[s * _NB1 + b], ciblk)
            _mask_block(dstbuf, dbuf2, lo, lo + _RNG1, _RNG1)
            _gather_scatter_block(embed_hbm, ciblk, 0, dbuf2,
                                  rows_v, rows_w, sem_a, sem_b, sem_c,
                                  sem_d, sem_e, onesv, acc, dega)
            return 0

        lax.fori_loop(0, _NB1, pass_body, 0)
        plsc.subcore_barrier()
        _drain(acc, dega, rows_v, degv, summed_hbm, deg_hbm, lo, s, _DR1)
        pltpu.sync_copy(zeros1_hbm, degv)
        plsc.subcore_barrier()

    wid = c * _NS + s
    nrow = _HD_W // _CH
    pltpu.sync_copy(nodes2d_hbm.at[wid], dbuf2.at[pl.ds(0, nrow)])
    for k in range(nrow):
        pltpu.sync_copy(embed_hbm.at[dbuf2.at[k]], rows_v)
        pltpu.sync_copy(rows_v,
                        hdst_hbm.at[pl.ds(wid * _HD_W + k * _CH, _CH)])


def _sc1_call(input_nodes, edge_src1, edge_dst1, embed):
    zeros = jnp.zeros((_L, _D), jnp.float32)
    zeros1 = jnp.zeros((_ACC1 // _NS,), jnp.float32)
    ones = jnp.ones((_CH,), jnp.float32)
    src3d = edge_src1.reshape(_NS * _NB1, _B // _CH, _CH)
    nodes2d = input_nodes[:_N_DST1].reshape(_NC * _NS, _HD_W // _CH, _CH)
    f = pl.kernel(
        _sc1_body,
        out_type=(
            jax.ShapeDtypeStruct((_N_DST1, _D), jnp.float32),
            jax.ShapeDtypeStruct((_N_DST1,), jnp.float32),
            jax.ShapeDtypeStruct((_N_DST1, _D), jnp.float32),
            jax.ShapeDtypeStruct((_NS * _NB1, _B // _CH, _CH), jnp.int32),
        ),
        mesh=_mesh,
        scratch_types=[
            pltpu.VMEM((_B // _CH, _CH), jnp.int32),
            pltpu.VMEM((_B,), jnp.int32),
            pltpu.VMEM((_B // _CH, _CH), jnp.int32),
            pltpu.VMEM((_CH, _D), jnp.float32),
            pltpu.VMEM((_CH, _D), jnp.float32),
            pltpu.VMEM((_CH,), jnp.float32),
            pltpu.VMEM((_L, _D), jnp.float32),
            pltpu.VMEM((_ACC1 // _NS,), jnp.float32),
            pltpu.VMEM_SHARED((_ACC1, _D), jnp.float32),
            pltpu.VMEM_SHARED((_ACC1,), jnp.float32),
            pltpu.SemaphoreType.DMA,
            pltpu.SemaphoreType.DMA,
            pltpu.SemaphoreType.DMA,
            pltpu.SemaphoreType.DMA,
            pltpu.SemaphoreType.DMA,
        ],
        compiler_params=_sc_params,
    )
    return f(input_nodes, src3d, edge_dst1, embed, nodes2d, zeros, zeros1,
             ones)


def _sc2_body(h_hbm, src3d_hbm, dst_hbm, zeros_hbm, zeros1_hbm, ones_hbm,
              summed_hbm, deg_hbm,
              srcbuf, dstbuf, dbuf2, rows_v, rows_w, onesv,
              zbuf, degv, acc, dega, sem_a, sem_b, sem_c, sem_d, sem_e):
    c = lax.axis_index("c")
    s = lax.axis_index("s")

    pltpu.sync_copy(zeros_hbm, zbuf)
    pltpu.sync_copy(ones_hbm, onesv)
    pltpu.sync_copy(zeros1_hbm, degv)
    pltpu.sync_copy(dst_hbm.at[pl.ds(s * _EW2, _EW2)], dstbuf)
    pltpu.sync_copy(src3d_hbm.at[s], srcbuf)

    lo = c * _HALF2
    _zero_acc(zbuf, degv, acc, dega, s)
    plsc.subcore_barrier()
    _mask_block(dstbuf, dbuf2, lo, lo + _HALF2, _HALF2)
    _gather_scatter_block(h_hbm, srcbuf, 0, dbuf2, rows_v, rows_w,
                          sem_a, sem_b, sem_c, sem_d, sem_e, onesv,
                          acc, dega)
    plsc.subcore_barrier()
    _drain(acc, dega, rows_v, degv, summed_hbm, deg_hbm, lo, s, _DR2)


def _sc2_call(h1, edge_src2, edge_dst2):
    zeros = jnp.zeros((_L, _D), jnp.float32)
    zeros1 = jnp.zeros((_ACC2 // _NS,), jnp.float32)
    ones = jnp.ones((_CH,), jnp.float32)
    f = pl.kernel(
        _sc2_body,
        out_type=(
            jax.ShapeDtypeStruct((_N_DST2, _D), jnp.float32),
            jax.ShapeDtypeStruct((_N_DST2,), jnp.float32),
        ),
        mesh=_mesh,
        scratch_types=[
            pltpu.VMEM((_EW2 // _CH, _CH), jnp.int32),
            pltpu.VMEM((_EW2,), jnp.int32),
            pltpu.VMEM((_B // _CH, _CH), jnp.int32),
            pltpu.VMEM((_CH, _D), jnp.float32),
            pltpu.VMEM((_CH, _D), jnp.float32),
            pltpu.VMEM((_CH,), jnp.float32),
            pltpu.VMEM((_L, _D), jnp.float32),
            pltpu.VMEM((_ACC2 // _NS,), jnp.float32),
            pltpu.VMEM_SHARED((_ACC2, _D), jnp.float32),
            pltpu.VMEM_SHARED((_ACC2,), jnp.float32),
            pltpu.SemaphoreType.DMA,
            pltpu.SemaphoreType.DMA,
            pltpu.SemaphoreType.DMA,
            pltpu.SemaphoreType.DMA,
            pltpu.SemaphoreType.DMA,
        ],
        compiler_params=_sc_params,
    )
    src3d = edge_src2.reshape(_NS, _EW2 // _CH, _CH)
    return f(h1, src3d, edge_dst2, zeros, zeros1, ones)


def _dense_body(relu, hd_ref, sm_ref, dg_ref, ws_ref, wn_ref, b_ref, o_ref):
    rd = 1.0 / jnp.maximum(dg_ref[...], 1.0)
    hn = sm_ref[...] * rd
    acc = jnp.dot(hd_ref[...], ws_ref[...], preferred_element_type=jnp.float32)
    acc += jnp.dot(hn, wn_ref[...], preferred_element_type=jnp.float32)
    acc += b_ref[...]
    if relu:
        acc = jnp.maximum(acc, 0.0)
    o_ref[...] = acc


def _dense_layer(h_dst, summed, deg, W_self, W_neigh, b, relu, block_m):
    m = h_dst.shape[0]
    grid = (m // block_m,)
    return pl.pallas_call(
        functools.partial(_dense_body, relu),
        grid=grid,
        in_specs=[
            pl.BlockSpec((block_m, _D), lambda i: (i, 0)),
            pl.BlockSpec((block_m, _D), lambda i: (i, 0)),
            pl.BlockSpec((block_m, 1), lambda i: (i, 0)),
            pl.BlockSpec((_D, _D), lambda i: (0, 0)),
            pl.BlockSpec((_D, _D), lambda i: (0, 0)),
            pl.BlockSpec((1, _D), lambda i: (0, 0)),
        ],
        out_specs=pl.BlockSpec((block_m, _D), lambda i: (i, 0)),
        out_shape=jax.ShapeDtypeStruct((m, _D), jnp.float32),
    )(h_dst, summed, deg.reshape(m, 1), W_self, W_neigh, b.reshape(1, _D))


def kernel(input_nodes, edge_src1, edge_dst1, edge_src2, edge_dst2, embed,
           W_self1, W_neigh1, b1, W_self2, W_neigh2, b2):
    summed1, deg1, h_dst1, _ = _sc1_call(input_nodes, edge_src1, edge_dst1,
                                         embed)
    out1 = _dense_layer(h_dst1, summed1, deg1, W_self1, W_neigh1, b1,
                        relu=True, block_m=512)
    summed2, deg2 = _sc2_call(out1, edge_src2, edge_dst2)
    out = _dense_layer(out1[:_N_DST2], summed2, deg2, W_self2, W_neigh2, b2,
                       relu=False, block_m=512)
    return out


# speedup vs baseline: 6.3689x; 1.5867x over previous
import functools

import jax
import jax.numpy as jnp
from jax import lax
from jax.experimental import pallas as pl
from jax.experimental.pallas import tpu as pltpu
from jax.experimental.pallas import tpu_sc as plsc

_D = 128
_N_NODES = 100000
_N_DST1 = 36864
_N_E1 = 294912
_N_DST2 = 4096
_N_E2 = 32768

_NC = 2
_NS = 16
_L = 16
_CH = 64
_CHS = 6
_B = 2048

_EW1 = _N_E1 // _NS
_NB1 = _EW1 // _B
_HALF1 = _N_DST1 // _NC
_NPASS1 = 2
_RNG1 = _HALF1 // _NPASS1
_ACC1 = 9344
_DR1 = _RNG1 // _NS
_HD_W = _N_DST1 // (_NC * _NS)

_EW2 = _N_E2 // _NS
_HALF2 = _N_DST2 // _NC
_ACC2 = 2176
_DR2 = _HALF2 // _NS

_mesh = plsc.VectorSubcoreMesh(core_axis_name="c", subcore_axis_name="s")
_sc_params = pltpu.CompilerParams(needs_layout_passes=False)


def _mask_block(dstbuf, dbuf2, lo, hi, trash):
    for r in range(_B // _CH):
        for j in range(_CH // _L):
            dv = dstbuf[pl.ds(r * _CH + j * _L, _L)]
            m = jnp.logical_and(dv >= lo, dv < hi)
            tvar = trash + jnp.bitwise_and(dv, 127)
            dbuf2[r, pl.ds(j * _L, _L)] = jnp.where(m, dv - lo, tvar)


def _gather_scatter_block(table_hbm, cibuf2d, cirow0, dbuf2, rows_v, rows_w,
                          sem_a, sem_b, sem_c, sem_d, sem_e, onesv,
                          acc, dega):
    rows = (rows_v, rows_w)
    gsem = (sem_a, sem_b)
    ssem = (sem_c, sem_d)
    nchb = _B // _CH
    gl = [
        pltpu.async_copy(table_hbm.at[cibuf2d.at[cirow0]], rows[0], gsem[0]),
        pltpu.async_copy(table_hbm.at[cibuf2d.at[cirow0 + 1]], rows[1],
                         gsem[1]),
    ]
    scs = [None] * nchb
    dcs = []
    for ch in range(nchb):
        gl[ch % 2].wait()
        scs[ch] = pltpu.async_copy(rows[ch % 2], acc.at[dbuf2.at[ch]],
                                   ssem[ch % 2], add=True)
        dcs.append(pltpu.async_copy(onesv, dega.at[dbuf2.at[ch]], sem_e,
                                    add=True))
        if ch + 2 < nchb:
            scs[ch].wait()
            gl[ch % 2] = pltpu.async_copy(
                table_hbm.at[cibuf2d.at[cirow0 + ch + 2]], rows[ch % 2],
                gsem[ch % 2])
    scs[nchb - 2].wait()
    scs[nchb - 1].wait()
    for cp in dcs:
        cp.wait()


def _zero_acc(zbuf, degzv, acc, dega, s):
    zw = dega.shape[0] // _NS
    base = s * zw
    for j in range(zw // _L):
        pltpu.sync_copy(zbuf, acc.at[pl.ds(base + j * _L, _L)])
    rem = zw - (zw // _L) * _L
    if rem:
        pltpu.sync_copy(zbuf.at[pl.ds(0, rem)],
                        acc.at[pl.ds(base + zw - rem, rem)])
    pltpu.sync_copy(degzv.at[pl.ds(0, zw)], dega.at[pl.ds(base, zw)])


def _drain(acc, dega, rows_v, degv, summed_hbm, deg_hbm, lo, s, dr):
    base = s * dr
    for k in range(dr // _CH):
        pltpu.sync_copy(acc.at[pl.ds(base + k * _CH, _CH)], rows_v)
        pltpu.sync_copy(rows_v, summed_hbm.at[pl.ds(lo + base + k * _CH, _CH)])
    rem = dr - (dr // _CH) * _CH
    if rem:
        pltpu.sync_copy(acc.at[pl.ds(base + dr - rem, rem)],
                        rows_v.at[pl.ds(0, rem)])
        pltpu.sync_copy(rows_v.at[pl.ds(0, rem)],
                        summed_hbm.at[pl.ds(lo + base + dr - rem, rem)])
    pltpu.sync_copy(dega.at[pl.ds(base, dr)], degv.at[pl.ds(0, dr)])
    pltpu.sync_copy(degv.at[pl.ds(0, dr)], deg_hbm.at[pl.ds(lo + base, dr)])


def _sc1_body(nodes_hbm, src3d_hbm, dst_hbm, embed_hbm, nodes2d_hbm,
              zeros_hbm, zeros1_hbm,
              ones_hbm, summed_hbm, deg_hbm, hdst_hbm, ci3d_hbm,
              ciblk, dstbuf, dbuf2, rows_v, rows_w, onesv, zbuf,
              degv, acc, dega, sem_a, sem_b, sem_c, sem_d, sem_e):
    c = lax.axis_index("c")
    s = lax.axis_index("s")

    pltpu.sync_copy(zeros_hbm, zbuf)
    pltpu.sync_copy(ones_hbm, onesv)
    pltpu.sync_copy(zeros1_hbm, degv)
    ebase = s * _EW1
    nchb = _B // _CH
    for b in range(_NB1):
        pltpu.sync_copy(src3d_hbm.at[s * _NB1 + b], dbuf2)
        cp = pltpu.async_copy(nodes_hbm.at[dbuf2.at[0]], ciblk.at[0], sem_a)
        for r in range(nchb):
            ncp = None
            if r + 1 < nchb:
                ncp = pltpu.async_copy(nodes_hbm.at[dbuf2.at[r + 1]],
                                       ciblk.at[r + 1],
                                       sem_b if (r + 1) % 2 else sem_a)
            cp.wait()
            cp = ncp
        pltpu.sync_copy(ciblk, ci3d_hbm.at[s * _NB1 + b])

    for p in range(_NPASS1):
        lo = c * _HALF1 + p * _RNG1
        _zero_acc(zbuf, degv, acc, dega, s)
        plsc.subcore_barrier()

        def pass_body(b, _, lo=lo):
            pltpu.sync_copy(dst_hbm.at[pl.ds(ebase + b * _B, _B)], dstbuf)
            pltpu.sync_copy(ci3d_hbm.at[s * _NB1 + b], ciblk)
            _mask_block(dstbuf, dbuf2, lo, lo + _RNG1, _RNG1)
            _gather_scatter_block(embed_hbm, ciblk, 0, dbuf2,
                                  rows_v, rows_w, sem_a, sem_b, sem_c,
                                  sem_d, sem_e, onesv, acc, dega)
            return 0

        lax.fori_loop(0, _NB1, pass_body, 0)
        plsc.subcore_barrier()
        _drain(acc, dega, rows_v, degv, summed_hbm, deg_hbm, lo, s, _DR1)
        pltpu.sync_copy(zeros1_hbm, degv)
        plsc.subcore_barrier()

    wid = c * _NS + s
    nrow = _HD_W // _CH
    pltpu.sync_copy(nodes2d_hbm.at[wid], dbuf2.at[pl.ds(0, nrow)])
    for k in range(nrow):
        pltpu.sync_copy(embed_hbm.at[dbuf2.at[k]], rows_v)
        pltpu.sync_copy(rows_v,
                        hdst_hbm.at[pl.ds(wid * _HD_W + k * _CH, _CH)])


def _sc1_call(input_nodes, edge_src1, edge_dst1, embed):
    zeros = jnp.zeros((_L, _D), jnp.float32)
    zeros1 = jnp.zeros((_ACC1 // _NS,), jnp.float32)
    ones = jnp.ones((_CH,), jnp.float32)
    src3d = edge_src1.reshape(_NS * _NB1, _B // _CH, _CH)
    nodes2d = input_nodes[:_N_DST1].reshape(_NC * _NS, _HD_W // _CH, _CH)
    f = pl.kernel(
        _sc1_body,
        out_type=(
            jax.ShapeDtypeStruct((_N_DST1, _D), jnp.float32),
            jax.ShapeDtypeStruct((_N_DST1,), jnp.float32),
            jax.ShapeDtypeStruct((_N_DST1, _D), jnp.float32),
            jax.ShapeDtypeStruct((_NS * _NB1, _B // _CH, _CH), jnp.int32),
        ),
        mesh=_mesh,
        scratch_types=[
            pltpu.VMEM((_B // _CH, _CH), jnp.int32),
            pltpu.VMEM((_B,), jnp.int32),
            pltpu.VMEM((_B // _CH, _CH), jnp.int32),
            pltpu.VMEM((_CH, _D), jnp.float32),
            pltpu.VMEM((_CH, _D), jnp.float32),
            pltpu.VMEM((_CH,), jnp.float32),
            pltpu.VMEM((_L, _D), jnp.float32),
            pltpu.VMEM((_ACC1 // _NS,), jnp.float32),
            pltpu.VMEM_SHARED((_ACC1, _D), jnp.float32),
            pltpu.VMEM_SHARED((_ACC1,), jnp.float32),
            pltpu.SemaphoreType.DMA,
            pltpu.SemaphoreType.DMA,
            pltpu.SemaphoreType.DMA,
            pltpu.SemaphoreType.DMA,
            pltpu.SemaphoreType.DMA,
        ],
        compiler_params=_sc_params,
    )
    return f(input_nodes, src3d, edge_dst1, embed, nodes2d, zeros, zeros1,
             ones)


def _sc2_body(h_hbm, src3d_hbm, dst_hbm, zeros_hbm, zeros1_hbm, ones_hbm,
              summed_hbm, deg_hbm,
              srcbuf, dstbuf, dbuf2, rows_v, rows_w, onesv,
              zbuf, degv, acc, dega, sem_a, sem_b, sem_c, sem_d, sem_e):
    c = lax.axis_index("c")
    s = lax.axis_index("s")

    pltpu.sync_copy(zeros_hbm, zbuf)
    pltpu.sync_copy(ones_hbm, onesv)
    pltpu.sync_copy(zeros1_hbm, degv)
    pltpu.sync_copy(dst_hbm.at[pl.ds(s * _EW2, _EW2)], dstbuf)
    pltpu.sync_copy(src3d_hbm.at[s], srcbuf)

    lo = c * _HALF2
    _zero_acc(zbuf, degv, acc, dega, s)
    plsc.subcore_barrier()
    _mask_block(dstbuf, dbuf2, lo, lo + _HALF2, _HALF2)
    _gather_scatter_block(h_hbm, srcbuf, 0, dbuf2, rows_v, rows_w,
                          sem_a, sem_b, sem_c, sem_d, sem_e, onesv,
                          acc, dega)
    plsc.subcore_barrier()
    _drain(acc, dega, rows_v, degv, summed_hbm, deg_hbm, lo, s, _DR2)


def _sc2_call(h1, edge_src2, edge_dst2):
    zeros = jnp.zeros((_L, _D), jnp.float32)
    zeros1 = jnp.zeros((_ACC2 // _NS,), jnp.float32)
    ones = jnp.ones((_CH,), jnp.float32)
    f = pl.kernel(
        _sc2_body,
        out_type=(
            jax.ShapeDtypeStruct((_N_DST2, _D), jnp.float32),
            jax.ShapeDtypeStruct((_N_DST2,), jnp.float32),
        ),
        mesh=_mesh,
        scratch_types=[
            pltpu.VMEM((_EW2 // _CH, _CH), jnp.int32),
            pltpu.VMEM((_EW2,), jnp.int32),
            pltpu.VMEM((_B // _CH, _CH), jnp.int32),
            pltpu.VMEM((_CH, _D), jnp.float32),
            pltpu.VMEM((_CH, _D), jnp.float32),
            pltpu.VMEM((_CH,), jnp.float32),
            pltpu.VMEM((_L, _D), jnp.float32),
            pltpu.VMEM((_ACC2 // _NS,), jnp.float32),
            pltpu.VMEM_SHARED((_ACC2, _D), jnp.float32),
            pltpu.VMEM_SHARED((_ACC2,), jnp.float32),
            pltpu.SemaphoreType.DMA,
            pltpu.SemaphoreType.DMA,
            pltpu.SemaphoreType.DMA,
            pltpu.SemaphoreType.DMA,
            pltpu.SemaphoreType.DMA,
        ],
        compiler_params=_sc_params,
    )
    src3d = edge_src2.reshape(_NS, _EW2 // _CH, _CH)
    return f(h1, src3d, edge_dst2, zeros, zeros1, ones)


def _dense_body(relu, hd_ref, sm_ref, dg_ref, ws_ref, wn_ref, b_ref, o_ref):
    rd = 1.0 / jnp.maximum(dg_ref[...], 1.0)
    hn = sm_ref[...] * rd
    acc = jnp.dot(hd_ref[...], ws_ref[...], preferred_element_type=jnp.float32)
    acc += jnp.dot(hn, wn_ref[...], preferred_element_type=jnp.float32)
    acc += b_ref[...]
    if relu:
        acc = jnp.maximum(acc, 0.0)
    o_ref[...] = acc


def _dense_layer(h_dst, summed, deg, W_self, W_neigh, b, relu, block_m):
    m = h_dst.shape[0]
    grid = (m // block_m,)
    return pl.pallas_call(
        functools.partial(_dense_body, relu),
        grid=grid,
        in_specs=[
            pl.BlockSpec((block_m, _D), lambda i: (i, 0)),
            pl.BlockSpec((block_m, _D), lambda i: (i, 0)),
            pl.BlockSpec((block_m, 1), lambda i: (i, 0)),
            pl.BlockSpec((_D, _D), lambda i: (0, 0)),
            pl.BlockSpec((_D, _D), lambda i: (0, 0)),
            pl.BlockSpec((1, _D), lambda i: (0, 0)),
        ],
        out_specs=pl.BlockSpec((block_m, _D), lambda i: (i, 0)),
        out_shape=jax.ShapeDtypeStruct((m, _D), jnp.float32),
    )(h_dst, summed, deg.reshape(m, 1), W_self, W_neigh, b.reshape(1, _D))


def kernel(input_nodes, edge_src1, edge_dst1, edge_src2, edge_dst2, embed,
           W_self1, W_neigh1, b1, W_self2, W_neigh2, b2):
    summed1, deg1, h_dst1, _ = _sc1_call(input_nodes, edge_src1, edge_dst1,
                                         embed)
    out1 = _dense_layer(h_dst1, summed1, deg1, W_self1, W_neigh1, b1,
                        relu=True, block_m=512)
    summed2, deg2 = _sc2_call(out1, edge_src2, edge_dst2)
    out = _dense_layer(out1[:_N_DST2], summed2, deg2, W_self2, W_neigh2, b2,
                       relu=False, block_m=512)
    return out
